# baseline (speedup 1.0000x reference)
import functools

import jax
import jax.numpy as jnp
from jax import lax
from jax.experimental import pallas as pl
from jax.experimental.pallas import tpu as pltpu
from jax.experimental.pallas import tpu_sc as plsc

HID = 64
NSEG = 16
NSTEPS = 50
EMB = 32
BN = 2048
BE = 4096

_NC = 2
_NS = 16


def _mm(a, b):
    return jax.lax.dot_general(a, b, (((1,), (0,)), ((), ())),
                               precision=jax.lax.Precision.DEFAULT)


def _wspec(shape):
    return pl.BlockSpec(shape, lambda i: (0,) * len(shape))


def _rows(bs, width):
    return pl.BlockSpec((bs, width), lambda i: (i, 0))


def _bd(w):
    z = jnp.zeros_like(w)
    return jnp.block([[w, z], [z, w]])


def _bb(b):
    return jnp.concatenate([b, b]).reshape(1, -1)


def _encode_nodes(X_t, node_features, rand_nodes, temb, ne, wpa, wpb, n_pad):
    (w1, b1), (w2, b2) = ne
    grid = (pl.cdiv(n_pad, BN),)

    def body(x_ref, nf_ref, rn_ref, te_ref, w1_ref, b1_ref, w2_ref, b2_ref,
             wpa_ref, wpb_ref, h_ref, p_ref):
        te = jnp.broadcast_to(te_ref[...], (x_ref.shape[0], EMB))
        x = jnp.concatenate([x_ref[...], nf_ref[...], te, rn_ref[...]],
                            axis=-1)
        hh = jnp.maximum(_mm(x, w1_ref[...]) + b1_ref[...], 0.0)
        h = _mm(hh, w2_ref[...]) + b2_ref[...]
        h_ref[...] = h
        p_ref[...] = jnp.concatenate(
            [_mm(h, wpa_ref[...]), _mm(h, wpb_ref[...])], axis=-1)

    return pl.pallas_call(
        body,
        grid=grid,
        in_specs=[_rows(BN, 2), _rows(BN, 2), _rows(BN, 5), _wspec((1, EMB)),
                  _wspec(w1.shape), _wspec((1, HID)), _wspec(w2.shape),
                  _wspec((1, HID)), _wspec((HID, HID)), _wspec((HID, HID))],
        out_specs=[_rows(BN, HID), _rows(BN, 2 * HID)],
        out_shape=[jax.ShapeDtypeStruct((n_pad, HID), jnp.float32),
                   jax.ShapeDtypeStruct((n_pad, 2 * HID), jnp.float32)],
    )(X_t, node_features, rand_nodes, temb, w1, b1.reshape(1, -1), w2,
      b2.reshape(1, -1), wpa, wpb)


def _edge_update(e2, g2, w1c2, b12, w22, b22):
    ep2 = g2.shape[0]

    def body(e_ref, g_ref, w1c_ref, b1_ref, w2_ref, b2_ref, out_ref):
        e = e_ref[...]
        hid = jnp.maximum(_mm(e, w1c_ref[...]) + g_ref[...] + b1_ref[...],
                          0.0)
        out_ref[...] = e + _mm(hid, w2_ref[...]) + b2_ref[...]

    return pl.pallas_call(
        body, grid=(ep2 // BE,),
        in_specs=[_rows(BE, 2 * HID), _rows(BE, 2 * HID),
                  _wspec((2 * HID, 2 * HID)), _wspec((1, 2 * HID)),
                  _wspec((2 * HID, 2 * HID)), _wspec((1, 2 * HID))],
        out_specs=_rows(BE, 2 * HID),
        out_shape=jax.ShapeDtypeStruct((ep2, 2 * HID), jnp.float32),
    )(e2, g2, w1c2, b12, w22, b22)


def _node_update(h, agg0, agg1, wh, wa0, wa1, bn1, wn2, bn2, wpa, wpb):
    ch = HID // _NC
    n_pad = h.shape[0]
    grid = (pl.cdiv(n_pad, BN),)

    def body(h_ref, a0_ref, a1_ref, wh_ref, wa0_ref, wa1_ref, b1_ref,
             w2_ref, b2_ref, wpa_ref, wpb_ref, hn_ref, p_ref):
        h = h_ref[...]
        hid = jnp.maximum(_mm(h, wh_ref[...]) + _mm(a0_ref[...], wa0_ref[...])
                          + _mm(a1_ref[...], wa1_ref[...]) + b1_ref[...], 0.0)
        hn = h + _mm(hid, w2_ref[...]) + b2_ref[...]
        hn_ref[...] = hn
        p_ref[...] = jnp.concatenate(
            [_mm(hn, wpa_ref[...]), _mm(hn, wpb_ref[...])], axis=-1)

    return pl.pallas_call(
        body, grid=grid,
        in_specs=[_rows(BN, HID), _rows(BN, ch), _rows(BN, ch),
                  _wspec((HID, HID)), _wspec((ch, HID)), _wspec((ch, HID)),
                  _wspec((1, HID)), _wspec((HID, HID)), _wspec((1, HID)),
                  _wspec((HID, HID)), _wspec((HID, HID))],
        out_specs=[_rows(BN, HID), _rows(BN, 2 * HID)],
        out_shape=[jax.ShapeDtypeStruct((n_pad, HID), jnp.float32),
                   jax.ShapeDtypeStruct((n_pad, 2 * HID), jnp.float32)],
    )(h, agg0, agg1, wh, wa0, wa1, bn1, wn2, bn2, wpa, wpb)


def _decode_heads(h, ngidx2d, n, dec, mh, lh):
    (dw1, db1), (dw2, db2) = dec
    (mw1, mb1), (mw2, mb2) = mh
    (lw1, lb1), (lw2, lb2) = lh
    grid = (pl.cdiv(n, BN),)

    def body(h_ref, ng_ref, dw1_ref, db1_ref, dw2_ref, db2_ref,
             mw1_ref, mb1_ref, mw2_ref, mb2_ref,
             lw1_ref, lb1_ref, lw2_ref, lb2_ref,
             mean_ref, logv_ref, vs_ref):
        i = pl.program_id(0)
        hh = jnp.maximum(_mm(h_ref[...], dw1_ref[...]) + db1_ref[...], 0.0)
        emb = _mm(hh, dw2_ref[...]) + db2_ref[...]
        mm = jnp.maximum(_mm(emb, mw1_ref[...]) + mb1_ref[...], 0.0)
        mean_ref[...] = _mm(mm, mw2_ref[...]) + mb2_ref[...]
        ll = jnp.maximum(_mm(emb, lw1_ref[...]) + lb1_ref[...], 0.0)
        logv_ref[...] = jnp.clip(_mm(ll, lw2_ref[...]) + lb2_ref[...],
                                 -10.0, 2.0)
        gids = jax.lax.broadcasted_iota(jnp.int32, (NSEG, BN), 0)
        col = jax.lax.broadcasted_iota(jnp.int32, (NSEG, BN), 1) + i * BN
        onehot = jnp.where((gids == ng_ref[...]) & (col < n), 1.0, 0.0)
        row = jax.lax.broadcasted_iota(jnp.int32, (BN, 1), 0) + i * BN
        emb_ext = jnp.concatenate(
            [emb, jnp.ones((BN, 1), jnp.float32)], axis=-1)
        emb_ext = jnp.where(row < n, emb_ext, 0.0)
        part = jnp.dot(onehot, emb_ext, preferred_element_type=jnp.float32,
                       precision=jax.lax.Precision.HIGHEST)

        @pl.when(i == 0)
        def _():
            vs_ref[...] = jnp.zeros_like(vs_ref)
        vs_ref[...] += part

    return pl.pallas_call(
        body, grid=grid,
        in_specs=[_rows(BN, HID), pl.BlockSpec((1, BN), lambda i: (0, i)),
                  _wspec((HID, HID)), _wspec((1, HID)), _wspec((HID, HID)),
                  _wspec((1, HID)),
                  _wspec((HID, HID)), _wspec((1, HID)), _wspec((HID, 2)),
                  _wspec((1, 2)),
                  _wspec((HID, HID)), _wspec((1, HID)), _wspec((HID, 2)),
                  _wspec((1, 2))],
        out_specs=[_rows(BN, 2), _rows(BN, 2),
                   pl.BlockSpec((NSEG, HID + 1), lambda i: (0, 0))],
        out_shape=[jax.ShapeDtypeStruct((n, 2), jnp.float32),
                   jax.ShapeDtypeStruct((n, 2), jnp.float32),
                   jax.ShapeDtypeStruct((NSEG, HID + 1), jnp.float32)],
    )(h, ngidx2d, dw1, db1.reshape(1, -1), dw2, db2.reshape(1, -1),
      mw1, mb1.reshape(1, -1), mw2, mb2.reshape(1, -1),
      lw1, lb1.reshape(1, -1), lw2, lb2.reshape(1, -1))


def _value_head(vs_ext, cnt_off, vh):
    (vw1, vb1), (vw2, vb2), (vw3, vb3) = vh

    def body(vs_ref, co_ref, w1_ref, b1_ref, w2_ref, b2_ref, w3_ref, b3_ref,
             out_ref):
        vs = vs_ref[:, :HID]
        counts = vs_ref[:, HID:HID + 1] + co_ref[...]
        vemb = vs / jnp.sqrt(jnp.clip(counts, 1.0, None))
        x = jnp.maximum(_mm(vemb, w1_ref[...]) + b1_ref[...], 0.0)
        x = jnp.maximum(_mm(x, w2_ref[...]) + b2_ref[...], 0.0)
        out_ref[...] = _mm(x, w3_ref[...]) + b3_ref[...]

    return pl.pallas_call(
        body,
        grid=(1,),
        in_specs=[_wspec((NSEG, HID + 1)), _wspec((1, 1)),
                  _wspec((HID, 120)), _wspec((1, 120)), _wspec((120, 64)),
                  _wspec((1, 64)), _wspec((64, 1)), _wspec((1, 1))],
        out_specs=_wspec((NSEG, 1)),
        out_shape=jax.ShapeDtypeStruct((NSEG, 1), jnp.float32),
    )(vs_ext, cnt_off, vw1, vb1.reshape(1, -1), vw2, vb2.reshape(1, -1),
      vw3, vb3.reshape(1, -1))


def _sc_gather_add(p_tab, src2d, dst2d):
    nrows = src2d.shape[0]
    per_w = nrows // (_NC * _NS)
    R = 2
    nsup = per_w // 8
    ep2 = nrows * 64
    mesh = plsc.VectorSubcoreMesh(core_axis_name="c", subcore_axis_name="s")

    @functools.partial(
        pl.kernel,
        out_type=jax.ShapeDtypeStruct((ep2, 2 * HID), jnp.float32),
        mesh=mesh,
        scratch_types=[
            pltpu.VMEM((8, 128), jnp.int32),
            pltpu.VMEM((8, 128), jnp.int32),
            pltpu.VMEM((R * 128, 2 * HID), jnp.float32),
            pltpu.VMEM((R * 128, 2 * HID), jnp.float32),
            pltpu.SemaphoreType.DMA,
            pltpu.SemaphoreType.DMA,
        ])
    def k(p_h, src_h, dst_h, out_h, ia, ib, ps, pd, sa, sb):
        w = lax.axis_index("s") * _NC + lax.axis_index("c")
        row0 = w * per_w

        @pl.loop(0, nsup)
        def _sup(gg):
            rb = row0 + gg * 8
            pltpu.sync_copy(src_h.at[pl.ds(rb, 8)], ia)
            pltpu.sync_copy(dst_h.at[pl.ds(rb, 8)], ib)
            for sub in range(8 // R):
                cps = []
                for j in range(R):
                    cps.append(pltpu.async_copy(
                        p_h.at[ia.at[sub * R + j]],
                        ps.at[pl.ds(j * 128, 128)], sa))
                    cps.append(pltpu.async_copy(
                        p_h.at[ib.at[sub * R + j]],
                        pd.at[pl.ds(j * 128, 128)], sb))
                for cp in cps:
                    cp.wait()

                @pl.loop(0, R * 64, unroll=4)
                def _row(j):
                    for half in range(2):
                        for colv in range(HID // 16):
                            dst_sl = pl.ds(half * HID + colv * 16, 16)
                            src_sl = pl.ds(colv * 16, 16)
                            add_sl = pl.ds(HID + colv * 16, 16)
                            ps[j, dst_sl] = (ps[2 * j + half, src_sl]
                                             + pd[2 * j + half, add_sl])

                pltpu.sync_copy(ps.at[pl.ds(0, R * 64)],
                                out_h.at[pl.ds((rb + sub * R) * 64, R * 64)])

    return k(p_tab, src2d, dst2d)


def _sc_scatter_sum(e2, sidx, n, n_pad):
    nrows = sidx[0][0].shape[0] // _NC
    per_t = nrows // _NS
    R = 2
    CH = HID // _NC
    q0 = 12504
    ztile = 792
    trows = _NS * ztile
    nhalf = n // 2
    mesh = plsc.VectorSubcoreMesh(core_axis_name="c", subcore_axis_name="s")

    @functools.partial(
        pl.kernel,
        out_type=[jax.ShapeDtypeStruct((n_pad, CH), jnp.float32),
                  jax.ShapeDtypeStruct((n_pad, CH), jnp.float32)],
        mesh=mesh,
        scratch_types=[
            pltpu.VMEM((8, 128), jnp.int32),
            pltpu.VMEM((8, 128), jnp.int32),
            pltpu.VMEM((R * 128, 2 * HID), jnp.float32),
            pltpu.VMEM((R * 128, CH), jnp.float32),
            pltpu.VMEM((R * 128, CH), jnp.float32),
            pltpu.VMEM_SHARED((trows, CH), jnp.float32),
            pltpu.SemaphoreType.DMA,
        ])
    def k(e_h, ie0_h, io0_h, ie1_h, io1_h, out0_h, out1_h, ie, io, full,
          bufe, bufo, table, sem):
        c = lax.axis_index("c")
        s = lax.axis_index("s")
        row0 = s * per_t

        for phc, phn in ((0, 0), (0, 1), (1, 0), (1, 1)):
            out_h = out0_h if phc == 0 else out1_h
            ie_h = ie0_h if phn == 0 else ie1_h
            io_h = io0_h if phn == 0 else io1_h

            @pl.loop(0, R * 128)
            def _z(row):
                for colv in range(CH // 16):
                    bufe[row, pl.ds(colv * 16, 16)] = jnp.zeros((16,),
                                                                jnp.float32)
            for zi in range(ztile // (R * 128)):
                pltpu.sync_copy(bufe,
                                table.at[pl.ds(s * ztile + zi * R * 128,
                                               R * 128)])
            rem = ztile % (R * 128)
            if rem:
                pltpu.sync_copy(bufe.at[pl.ds(0, rem)],
                                table.at[pl.ds((s + 1) * ztile - rem, rem)])
            plsc.subcore_barrier()

            @pl.loop(0, per_t // 8)
            def _sup(gg):
                rb = row0 + gg * 8
                pltpu.sync_copy(ie_h.at[pl.ds(c * nrows + rb, 8)], ie)
                pltpu.sync_copy(io_h.at[pl.ds(c * nrows + rb, 8)], io)
                for sub in range(8 // R):
                    r = rb + sub * R
                    pltpu.sync_copy(e_h.at[pl.ds(r * 128, R * 128)], full)

                    @pl.loop(0, R * 128, unroll=4)
                    def _x(row):
                        for colv in range(CH // 16):
                            bufe[row, pl.ds(colv * 16, 16)] = (
                                full[row, pl.ds(phc * CH + colv * 16, 16)])
                            bufo[row, pl.ds(colv * 16, 16)] = (
                                full[row, pl.ds(HID + phc * CH + colv * 16,
                                                16)])

                    for j in range(R):
                        pltpu.sync_copy(bufe.at[pl.ds(j * 128, 128)],
                                        table.at[ie.at[sub * R + j]],
                                        add=True)
                        pltpu.sync_copy(bufo.at[pl.ds(j * 128, 128)],
                                        table.at[io.at[sub * R + j]],
                                        add=True)

            plsc.subcore_barrier()

            fs = 784
            for cc in (0, 1):
                span = (q0 if cc == 0 else nhalf - q0
                        + ((n_pad - n) if phn == 1 else 0))
                span_last = span - (_NS - 1) * fs
                qb = phn * nhalf + cc * q0

                @pl.when((c == cc) & (s < _NS - 1))
                def _f():
                    pltpu.sync_copy(table.at[pl.ds(s * fs, fs)],
                                    out_h.at[pl.ds(qb + s * fs, fs)])

                @pl.when((c == cc) & (s == _NS - 1))
                def _fl():
                    pltpu.sync_copy(
                        table.at[pl.ds((_NS - 1) * fs, span_last)],
                        out_h.at[pl.ds(qb + (_NS - 1) * fs, span_last)])

            plsc.subcore_barrier()

    return k(e2, sidx[0][0], sidx[0][1], sidx[1][0], sidx[1][1])


def _edge_encode(attr2, enc):
    (ew1, eb1), (ew2, eb2) = enc
    ew12, eb12, ew22, eb22 = _bd(ew1), _bb(eb1), _bd(ew2), _bb(eb2)
    ep2 = attr2.shape[0]

    def body(a_ref, w1_ref, b1_ref, w2_ref, b2_ref, out_ref):
        eh = jnp.maximum(_mm(a_ref[...], w1_ref[...]) + b1_ref[...], 0.0)
        out_ref[...] = _mm(eh, w2_ref[...]) + b2_ref[...]

    return pl.pallas_call(
        body, grid=(ep2 // BE,),
        in_specs=[_rows(BE, 8), _wspec((8, 2 * HID)), _wspec((1, 2 * HID)),
                  _wspec((2 * HID, 2 * HID)), _wspec((1, 2 * HID))],
        out_specs=_rows(BE, 2 * HID),
        out_shape=jax.ShapeDtypeStruct((ep2, 2 * HID), jnp.float32),
    )(attr2, ew12, eb12, ew22, eb22)


def kernel(X_t, t_idx, edge_index, edge_attr, node_graph_idx, n_graphs,
           node_features, rand_nodes, params):
    n = X_t.shape[0]
    e_cnt = edge_index.shape[1]
    n_pad = n + 8
    ep = ((e_cnt + 32 * 128 * 8 - 1) // (32 * 128 * 8)) * (32 * 128 * 8)
    ep2 = ep // 2

    half = EMB // 2
    freqs = jnp.exp(-jnp.log(float(NSTEPS))
                    * jnp.arange(half, dtype=jnp.float32) / half)
    targs = jnp.asarray(t_idx, jnp.float32) * freqs
    temb = jnp.concatenate([jnp.sin(targs), jnp.cos(targs)]).reshape(1, EMB)

    src_pad = jnp.pad(edge_index[0], (0, ep - e_cnt), constant_values=n)
    dst_pad = jnp.pad(edge_index[1], (0, ep - e_cnt), constant_values=n)
    src2d = src_pad.reshape(ep // 128, 128)
    dst2d = dst_pad.reshape(ep // 128, 128)
    dpair = dst_pad.reshape(ep2, 2)
    attr2 = jnp.pad(edge_attr, ((0, ep - e_cnt), (0, 0))).reshape(ep2, 8)

    q0, trash = 12504, 12560
    nhalf = n // 2

    def _remap(d, phn):
        rows = []
        for c in (0, 1):
            rel = d - (phn * nhalf + c * q0)
            ok = (rel >= 0) & (rel < q0 - c * 8)
            rows.append(jnp.where(ok, rel, trash).reshape(-1, 128))
        return jnp.concatenate(rows)

    sidx = tuple((_remap(dpair[:, 0], phn), _remap(dpair[:, 1], phn))
                 for phn in (0, 1))

    p = params
    wa, wb, wc2, b12, w22, b22 = [], [], [], [], [], []
    wh, wa0, wa1, bn1, wn2, bn2 = [], [], [], [], [], []
    ch = HID // _NC
    for pp in p['passes']:
        (w1, b1), (w2, b2) = pp['edge_mlp']
        wa.append(w1[:HID])
        wb.append(w1[HID:2 * HID])
        wc2.append(_bd(w1[2 * HID:]))
        b12.append(_bb(b1))
        w22.append(_bd(w2))
        b22.append(_bb(b2))
        (nw1, nb1), (nw2, nb2) = pp['node_mlp']
        wh.append(nw1[:HID])
        wa0.append(nw1[HID:HID + ch])
        wa1.append(nw1[HID + ch:])
        bn1.append(nb1.reshape(1, -1))
        wn2.append(nw2)
        bn2.append(nb2.reshape(1, -1))
    wpa = jnp.stack(wa[1:] + [jnp.zeros((HID, HID), jnp.float32)])
    wpb = jnp.stack(wb[1:] + [jnp.zeros((HID, HID), jnp.float32)])
    xs = (jnp.stack(wc2), jnp.stack(b12), jnp.stack(w22), jnp.stack(b22),
          jnp.stack(wh), jnp.stack(wa0), jnp.stack(wa1), jnp.stack(bn1),
          jnp.stack(wn2), jnp.stack(bn2), wpa, wpb)

    h, ptab = _encode_nodes(X_t, node_features, rand_nodes, temb,
                            p['node_enc'], wa[0], wb[0], n_pad)
    e2 = _edge_encode(attr2, p['edge_enc'])

    def step(carry, x):
        h, ptab, e2 = carry
        (wc2_, b12_, w22_, b22_, wh_, wa0_, wa1_, bn1_, wn2_, bn2_,
         wpa_, wpb_) = x
        g2 = _sc_gather_add(ptab, src2d, dst2d)
        e2n = _edge_update(e2, g2, wc2_, b12_, w22_, b22_)
        ef = e2n.reshape(ep, HID)
        agg = jax.ops.segment_sum(ef, dst_pad, num_segments=n + 1)
        agg = jnp.pad(agg[:n], ((0, n_pad - n), (0, 0)))
        agg0, agg1 = agg[:, :32], agg[:, 32:]
        hn, ptabn = _node_update(h, agg0, agg1, wh_, wa0_, wa1_, bn1_,
                                 wn2_, bn2_, wpa_, wpb_)
        return (hn, ptabn, e2n), None

    (h, _, _), _ = lax.scan(step, (h, ptab, e2), xs)

    ngidx2d = node_graph_idx.reshape(1, n).astype(jnp.int32)
    mean, logv, vs_ext = _decode_heads(h, ngidx2d, n, p['decoder'],
                                       p['mean_head'], p['log_var_head'])
    cnt_off = (jnp.asarray(n_graphs, jnp.float32) - float(NSEG)).reshape(1, 1)
    values = _value_head(vs_ext, cnt_off, p['value_head']).reshape(NSEG)
    return (mean, logv, values, rand_nodes)
